# baseline (speedup 1.0000x reference)
import functools

import jax
import jax.numpy as jnp
from jax.experimental import pallas as pl
from jax.experimental.pallas import tpu as pltpu

F32 = jnp.float32
_EPS = 1e-5
_LANES = 128


def _lane_mask(d):
    iota = jax.lax.broadcasted_iota(jnp.int32, (1, _LANES), 1)
    return iota == jax.lax.rem(d, _LANES)


def _packed_get(ref, d):
    row = ref[pl.ds(jax.lax.div(d, _LANES), 1), :]
    return jnp.sum(jnp.where(_lane_mask(d), row, 0.0), axis=1, keepdims=True)


def _packed_rmw(ref, d, val, op):
    r = jax.lax.div(d, _LANES)
    row = ref[pl.ds(r, 1), :]
    ref[pl.ds(r, 1), :] = jnp.where(_lane_mask(d), op(row, val), row)

_SEQ = pltpu.CompilerParams(dimension_semantics=("arbitrary",))


def _full(shape):
    return pl.BlockSpec(shape, lambda i: tuple(0 for _ in shape))


def _rows(b, cols):
    return pl.BlockSpec((b, cols), lambda i: (i, 0))


def _lane(c):
    return pl.BlockSpec((1, c), lambda i: (0, i))


def _lane_smem(c):
    return pl.BlockSpec((1, c), lambda i: (0, i), memory_space=pltpu.SMEM)


def _stats_kernel(x_ref, o_ref):
    @pl.when(pl.program_id(0) == 0)
    def _():
        o_ref[...] = jnp.zeros_like(o_ref)

    xb = x_ref[...]
    o_ref[0:1, :] += jnp.sum(xb, axis=0, keepdims=True)
    o_ref[1:2, :] += jnp.sum(xb * xb, axis=0, keepdims=True)


def _col_stats(x, b):
    n, f = x.shape
    return pl.pallas_call(
        _stats_kernel,
        grid=(n // b,),
        in_specs=[_rows(b, f)],
        out_specs=_full((8, f)),
        out_shape=jax.ShapeDtypeStruct((8, f), F32),
        compiler_params=_SEQ,
    )(x)


def _proj_kernel(x_ref, st_ref, g_ref, b_ref, wq_ref, bq_ref, wk_ref, bk_ref,
                 wv_ref, bv_ref, ws_ref, bs_ref, q_ref, k_ref, v_ref, hs_ref,
                 *, n):
    st = st_ref[...]
    mu = st[0:1, :] / n
    var = st[1:2, :] / n - mu * mu
    h = (x_ref[...] - mu) * jax.lax.rsqrt(var + _EPS) * g_ref[...] + b_ref[...]
    q_ref[...] = jnp.dot(h, wq_ref[...], preferred_element_type=F32) + bq_ref[...]
    k_ref[...] = jnp.dot(h, wk_ref[...], preferred_element_type=F32) + bk_ref[...]
    v_ref[...] = jnp.dot(h, wv_ref[...], preferred_element_type=F32) + bv_ref[...]
    hs_ref[...] = jnp.dot(h, ws_ref[...], preferred_element_type=F32) + bs_ref[...]


def _att1_kernel(src_ref, dst_ref, ea_ref, q_ref, k_ref, we_ref, be_ref,
                 lg_ref, mx_ref, ks, qd, *, c, h):
    @pl.when(pl.program_id(0) == 0)
    def _():
        mx_ref[...] = jnp.full_like(mx_ref, -jnp.inf)

    def gather(j, carry):
        s = src_ref[0, j]
        d = dst_ref[0, j]
        ks[pl.ds(j, 1), :] = k_ref[pl.ds(s, 1), :]
        qd[pl.ds(j, 1), :] = q_ref[pl.ds(d, 1), :]
        return carry

    jax.lax.fori_loop(0, c, gather, 0)
    e = jnp.dot(ea_ref[...], we_ref[...], preferred_element_type=F32) + be_ref[...]
    lg = jnp.sum(qd[...] * (ks[...] + e), axis=1, keepdims=True)
    lg_ref[...] = lg / jnp.sqrt(jnp.float32(h))

    def smax(j, carry):
        d = dst_ref[0, j]
        _packed_rmw(mx_ref, d, lg_ref[pl.ds(j, 1), 0:1], jnp.maximum)
        return carry

    jax.lax.fori_loop(0, c, smax, 0)


def _att2_kernel(dst_ref, lg_ref, mx_ref, ex_ref, s_ref, md, *, c):
    @pl.when(pl.program_id(0) == 0)
    def _():
        s_ref[...] = jnp.zeros_like(s_ref)

    def gmax(j, carry):
        d = dst_ref[0, j]
        md[pl.ds(j, 1), 0:1] = _packed_get(mx_ref, d)
        return carry

    jax.lax.fori_loop(0, c, gmax, 0)
    ex_ref[...] = jnp.exp(lg_ref[...] - md[...])

    def sadd(j, carry):
        d = dst_ref[0, j]
        _packed_rmw(s_ref, d, ex_ref[pl.ds(j, 1), 0:1], jnp.add)
        return carry

    jax.lax.fori_loop(0, c, sadd, 0)


def _att3_kernel(src_ref, dst_ref, ea_ref, ex_ref, v_ref, s_ref, we_ref,
                 be_ref, acc_ref, rows, wts, *, c):
    @pl.when(pl.program_id(0) == 0)
    def _():
        acc_ref[...] = jnp.zeros_like(acc_ref)

    def gather(j, carry):
        s = src_ref[0, j]
        d = dst_ref[0, j]
        rows[pl.ds(j, 1), :] = v_ref[pl.ds(s, 1), :]
        wts[pl.ds(j, 1), 0:1] = _packed_get(s_ref, d)
        return carry

    jax.lax.fori_loop(0, c, gather, 0)
    e = jnp.dot(ea_ref[...], we_ref[...], preferred_element_type=F32) + be_ref[...]
    alpha = ex_ref[...] / wts[...]
    rows[...] = (rows[...] + e) * alpha

    def scat(j, carry):
        d = dst_ref[0, j]
        acc_ref[pl.ds(d, 1), :] += rows[pl.ds(j, 1), :]
        return carry

    jax.lax.fori_loop(0, c, scat, 0)


def _addmat_kernel(a_ref, b_ref, w_ref, h_ref, t_ref):
    hv = a_ref[...] + b_ref[...]
    h_ref[...] = hv
    t_ref[...] = jnp.dot(hv, w_ref[...], preferred_element_type=F32)


def _mat_kernel(a_ref, w_ref, t_ref):
    t_ref[...] = jnp.dot(a_ref[...], w_ref[...], preferred_element_type=F32)


def _segsum_kernel(src_ref, dst_ref, t_ref, o_ref, *, c):
    @pl.when(pl.program_id(0) == 0)
    def _():
        o_ref[...] = jnp.zeros_like(o_ref)

    def scat(j, carry):
        s = src_ref[0, j]
        d = dst_ref[0, j]
        o_ref[pl.ds(d, 1), :] += t_ref[pl.ds(s, 1), :]
        return carry

    jax.lax.fori_loop(0, c, scat, 0)


def _segsum(src2d, dst2d, table, n_out, c):
    e = src2d.shape[1]
    nt, hd = table.shape
    return pl.pallas_call(
        functools.partial(_segsum_kernel, c=c),
        grid=(e // c,),
        in_specs=[_lane_smem(c), _lane_smem(c), _full((nt, hd))],
        out_specs=_full((n_out, hd)),
        out_shape=jax.ShapeDtypeStruct((n_out, hd), F32),
        compiler_params=_SEQ,
    )(src2d, dst2d, table)


def _gru_kernel(m_ref, h_ref, wi_ref, wh_ref, bi_ref, bh_ref, o_ref, *, hd):
    gi = jnp.dot(m_ref[...], wi_ref[...], preferred_element_type=F32) + bi_ref[...]
    gh = jnp.dot(h_ref[...], wh_ref[...], preferred_element_type=F32) + bh_ref[...]
    hv = h_ref[...]
    r = jax.nn.sigmoid(gi[:, :hd] + gh[:, :hd])
    z = jax.nn.sigmoid(gi[:, hd:2 * hd] + gh[:, hd:2 * hd])
    cc = jnp.tanh(gi[:, 2 * hd:] + r * gh[:, 2 * hd:])
    o_ref[...] = (1.0 - z) * cc + z * hv


def _gru(m, h, wiT, whT, bi2, bh2, b):
    n, hd = h.shape
    return pl.pallas_call(
        functools.partial(_gru_kernel, hd=hd),
        grid=(n // b,),
        in_specs=[_rows(b, hd), _rows(b, hd), _full(wiT.shape), _full(whT.shape),
                  _full(bi2.shape), _full(bh2.shape)],
        out_specs=_rows(b, hd),
        out_shape=jax.ShapeDtypeStruct((n, hd), F32),
        compiler_params=_SEQ,
    )(m, h, wiT, whT, bi2, bh2)


def _ggc_stack(h, src2d, dst2d, Ws, wiT, whT, bi2, bh2, n_out, c, b,
               skip=None):
    n, hd = h.shape
    for i, W in enumerate(Ws):
        if i == 0 and skip is not None:
            h, t = pl.pallas_call(
                _addmat_kernel,
                grid=(n // b,),
                in_specs=[_rows(b, hd), _rows(b, hd), _full((hd, hd))],
                out_specs=[_rows(b, hd), _rows(b, hd)],
                out_shape=[jax.ShapeDtypeStruct((n, hd), F32),
                           jax.ShapeDtypeStruct((n, hd), F32)],
                compiler_params=_SEQ,
            )(h, skip, W)
        else:
            t = pl.pallas_call(
                _mat_kernel,
                grid=(n // b,),
                in_specs=[_rows(b, hd), _full((hd, hd))],
                out_specs=_rows(b, hd),
                out_shape=jax.ShapeDtypeStruct((n, hd), F32),
                compiler_params=_SEQ,
            )(h, W)
        m = _segsum(src2d, dst2d, t, n_out, c)
        h = _gru(m, h, wiT, whT, bi2, bh2, b)
    return h


def _pool_kernel(idx_ref, h_ref, o_ref, cnt_ref, *, b):
    @pl.when(pl.program_id(0) == 0)
    def _():
        o_ref[...] = jnp.zeros_like(o_ref)
        cnt_ref[...] = jnp.zeros_like(cnt_ref)

    base = pl.program_id(0) * b

    def scat(j, carry):
        d = idx_ref[0, base + j]
        o_ref[pl.ds(d, 1), :] += h_ref[pl.ds(j, 1), :]
        cnt_ref[pl.ds(d, 1), 0:1] += 1.0
        return carry

    jax.lax.fori_loop(0, b, scat, 0)


def _mean_stats_kernel(s_ref, c_ref, o_ref, st_ref):
    @pl.when(pl.program_id(0) == 0)
    def _():
        st_ref[...] = jnp.zeros_like(st_ref)

    hv = s_ref[...] / jnp.maximum(c_ref[...], 1.0)
    o_ref[...] = hv
    st_ref[0:1, :] += jnp.sum(hv, axis=0, keepdims=True)
    st_ref[1:2, :] += jnp.sum(hv * hv, axis=0, keepdims=True)


def _head1_kernel(pid_ref, h_ref, st_ref, g_ref, b_ref, et_ref, wp1_ref,
                  wp2_ref, bp_ref, o_ref, emb, *, b, na):
    st = st_ref[...]
    mu = st[0:1, :] / na
    var = st[1:2, :] / na - mu * mu
    hv = (h_ref[...] - mu) * jax.lax.rsqrt(var + _EPS) * g_ref[...] + b_ref[...]

    base = pl.program_id(0) * b

    def gather(j, carry):
        p = pid_ref[0, base + j]
        emb[pl.ds(j, 1), :] = et_ref[pl.ds(p, 1), :]
        return carry

    jax.lax.fori_loop(0, b, gather, 0)
    o_ref[...] = jax.nn.relu(
        jnp.dot(hv, wp1_ref[...], preferred_element_type=F32)
        + jnp.dot(emb[...], wp2_ref[...], preferred_element_type=F32)
        + bp_ref[...])


def _head2_kernel(i0_ref, i1_ref, h_ref, w1a_ref, w1b_ref, b1_ref, w2_ref,
                  b2_ref, o_ref, rl, rr, *, c):
    def gather(j, carry):
        a = i0_ref[0, j]
        bb = i1_ref[0, j]
        rl[pl.ds(j, 1), :] = h_ref[pl.ds(a, 1), :]
        rr[pl.ds(j, 1), :] = h_ref[pl.ds(bb, 1), :]
        return carry

    jax.lax.fori_loop(0, c, gather, 0)
    hh = jax.nn.relu(
        jnp.dot(rl[...], w1a_ref[...], preferred_element_type=F32)
        + jnp.dot(rr[...], w1b_ref[...], preferred_element_type=F32)
        + b1_ref[...])
    o_ref[...] = (jnp.dot(hh, w2_ref[...], preferred_element_type=F32)
                  + b2_ref[...])


def kernel(x, inner_edge_index, edge_attr, aminoacid_index, protease_id,
           edge_index, gamma1, beta1, Wq, bq, Wk, bk, Wv, bv, We, be, Wskip,
           bskip, Wg1, Wih1, Whh1, bih1, bhh1, gamma2, beta2, emb_table, Wp,
           bp, Wg2, Wih2, Whh2, bih2, bhh2, Wm1, bm1, Wm2, bm2):
    n, f = x.shape
    e_in = inner_edge_index.shape[1]
    na = protease_id.shape[0]
    hd = Wq.shape[1]
    pe = emb_table.shape[1]
    e_out = edge_index.shape[1]
    npairs = e_out // 2

    def _pick(total, prefs):
        for p in prefs:
            if total % p == 0:
                return p
        return total

    nr = (((n + _LANES - 1) // _LANES) + 7) // 8 * 8
    NB = _pick(n, [1000, 500, 250, 200, 100])
    NBa = _pick(na, [1000, 500, 250, 200, 100])
    CE = _pick(e_in, [640, 1280])
    CO = _pick(e_out, [640, 1280])
    CP = _pick(npairs, [3200, 640])

    x = x.astype(F32)
    src = inner_edge_index[0].reshape(1, e_in)
    dst = inner_edge_index[1].reshape(1, e_in)
    osrc = edge_index[0].reshape(1, e_out)
    odst = edge_index[1].reshape(1, e_out)
    i0 = edge_index[0, ::2].reshape(1, npairs)
    i1 = edge_index[1, ::2].reshape(1, npairs)
    aa = aminoacid_index.reshape(1, n)
    pid = protease_id.reshape(1, na)

    row = lambda v: v.reshape(1, -1).astype(F32)

    st1 = _col_stats(x, NB)
    q, k, v, hskip = pl.pallas_call(
        functools.partial(_proj_kernel, n=float(n)),
        grid=(n // NB,),
        in_specs=[_rows(NB, f), _full((8, f)), _full((1, f)), _full((1, f)),
                  _full((f, hd)), _full((1, hd)), _full((f, hd)), _full((1, hd)),
                  _full((f, hd)), _full((1, hd)), _full((f, hd)), _full((1, hd))],
        out_specs=[_rows(NB, hd)] * 4,
        out_shape=[jax.ShapeDtypeStruct((n, hd), F32)] * 4,
        compiler_params=_SEQ,
    )(x, st1, row(gamma1), row(beta1), Wq, row(bq), Wk, row(bk), Wv, row(bv),
      Wskip, row(bskip))

    lg, mx = pl.pallas_call(
        functools.partial(_att1_kernel, c=CE, h=hd),
        grid=(e_in // CE,),
        in_specs=[_lane_smem(CE), _lane_smem(CE),
                  pl.BlockSpec((CE, 5), lambda i: (i, 0)),
                  _full((n, hd)), _full((n, hd)), _full((5, hd)), _full((1, hd))],
        out_specs=[pl.BlockSpec((CE, 1), lambda i: (i, 0)), _full((nr, _LANES))],
        out_shape=[jax.ShapeDtypeStruct((e_in, 1), F32),
                   jax.ShapeDtypeStruct((nr, _LANES), F32)],
        scratch_shapes=[pltpu.VMEM((CE, hd), F32), pltpu.VMEM((CE, hd), F32)],
        compiler_params=_SEQ,
    )(src, dst, edge_attr, q, k, We, row(be))

    ex, ssum = pl.pallas_call(
        functools.partial(_att2_kernel, c=CE),
        grid=(e_in // CE,),
        in_specs=[_lane_smem(CE), pl.BlockSpec((CE, 1), lambda i: (i, 0)),
                  _full((nr, _LANES))],
        out_specs=[pl.BlockSpec((CE, 1), lambda i: (i, 0)),
                   _full((nr, _LANES))],
        out_shape=[jax.ShapeDtypeStruct((e_in, 1), F32),
                   jax.ShapeDtypeStruct((nr, _LANES), F32)],
        scratch_shapes=[pltpu.VMEM((CE, 1), F32)],
        compiler_params=_SEQ,
    )(dst, lg, mx)

    agg = pl.pallas_call(
        functools.partial(_att3_kernel, c=CE),
        grid=(e_in // CE,),
        in_specs=[_lane_smem(CE), _lane_smem(CE),
                  pl.BlockSpec((CE, 5), lambda i: (i, 0)),
                  pl.BlockSpec((CE, 1), lambda i: (i, 0)), _full((n, hd)),
                  _full((nr, _LANES)), _full((5, hd)), _full((1, hd))],
        out_specs=_full((n, hd)),
        out_shape=jax.ShapeDtypeStruct((n, hd), F32),
        scratch_shapes=[pltpu.VMEM((CE, hd), F32), pltpu.VMEM((CE, 1), F32)],
        compiler_params=_SEQ,
    )(src, dst, edge_attr, ex, v, ssum, We, row(be))

    h = _ggc_stack(agg, src, dst, [Wg1[i] for i in range(Wg1.shape[0])],
                   Wih1.T, Whh1.T, row(bih1), row(bhh1), n, CE, NB,
                   skip=hskip)

    sums, cnt = pl.pallas_call(
        functools.partial(_pool_kernel, b=NB),
        grid=(n // NB,),
        in_specs=[pl.BlockSpec((1, n), lambda i: (0, 0), memory_space=pltpu.SMEM),
                  _rows(NB, hd)],
        out_specs=[_full((na, hd)), _full((na, 1))],
        out_shape=[jax.ShapeDtypeStruct((na, hd), F32),
                   jax.ShapeDtypeStruct((na, 1), F32)],
        compiler_params=_SEQ,
    )(aa, h)

    hp, st2 = pl.pallas_call(
        _mean_stats_kernel,
        grid=(na // NBa,),
        in_specs=[_rows(NBa, hd), pl.BlockSpec((NBa, 1), lambda i: (i, 0))],
        out_specs=[_rows(NB, hd), _full((8, hd))],
        out_shape=[jax.ShapeDtypeStruct((na, hd), F32),
                   jax.ShapeDtypeStruct((8, hd), F32)],
        compiler_params=_SEQ,
    )(sums, cnt)

    h2 = pl.pallas_call(
        functools.partial(_head1_kernel, b=NBa, na=float(na)),
        grid=(na // NBa,),
        in_specs=[pl.BlockSpec((1, na), lambda i: (0, 0), memory_space=pltpu.SMEM),
                  _rows(NBa, hd), _full((8, hd)),
                  _full((1, hd)), _full((1, hd)), _full(emb_table.shape),
                  _full((hd, hd)), _full((pe, hd)), _full((1, hd))],
        out_specs=_rows(NBa, hd),
        out_shape=jax.ShapeDtypeStruct((na, hd), F32),
        scratch_shapes=[pltpu.VMEM((NBa, pe), F32)],
        compiler_params=_SEQ,
    )(pid, hp, st2, row(gamma2), row(beta2), emb_table, Wp[:hd], Wp[hd:],
      row(bp))

    h2 = _ggc_stack(h2, osrc, odst, [Wg2[i] for i in range(Wg2.shape[0])],
                    Wih2.T, Whh2.T, row(bih2), row(bhh2), na, CO, NBa)

    out = pl.pallas_call(
        functools.partial(_head2_kernel, c=CP),
        grid=(npairs // CP,),
        in_specs=[_lane_smem(CP), _lane_smem(CP), _full((na, hd)),
                  _full((hd, hd)), _full((hd, hd)), _full((1, hd)),
                  _full((hd, 1)), _full((1, 1))],
        out_specs=pl.BlockSpec((CP, 1), lambda i: (i, 0)),
        out_shape=jax.ShapeDtypeStruct((npairs, 1), F32),
        scratch_shapes=[pltpu.VMEM((CP, hd), F32), pltpu.VMEM((CP, hd), F32)],
        compiler_params=_SEQ,
    )(i0, i1, h2, Wm1[:hd], Wm1[hd:], row(bm1), Wm2, row(bm2))

    return out.reshape(-1)


# speedup vs baseline: 5.1753x; 5.1753x over previous
import functools

import jax
import jax.numpy as jnp
from jax.experimental import pallas as pl
from jax.experimental.pallas import tpu as pltpu

F32 = jnp.float32
_EPS = 1e-5
_LANES = 128


def _lane_mask(d):
    iota = jax.lax.broadcasted_iota(jnp.int32, (1, _LANES), 1)
    return iota == jax.lax.rem(d, _LANES)


def _packed_get(ref, d):
    row = ref[pl.ds(jax.lax.div(d, _LANES), 1), :]
    return jnp.sum(jnp.where(_lane_mask(d), row, 0.0), axis=1, keepdims=True)


def _packed_rmw(ref, d, val, op):
    r = jax.lax.div(d, _LANES)
    row = ref[pl.ds(r, 1), :]
    ref[pl.ds(r, 1), :] = jnp.where(_lane_mask(d), op(row, val), row)

_SEQ = pltpu.CompilerParams(dimension_semantics=("arbitrary",))


def _full(shape):
    return pl.BlockSpec(shape, lambda i: tuple(0 for _ in shape))


def _rows(b, cols):
    return pl.BlockSpec((b, cols), lambda i: (i, 0))


def _lane(c):
    return pl.BlockSpec((1, c), lambda i: (0, i))


def _lane_smem(c):
    return pl.BlockSpec((1, c), lambda i: (0, i), memory_space=pltpu.SMEM)


def _stats_kernel(x_ref, o_ref):
    @pl.when(pl.program_id(0) == 0)
    def _():
        o_ref[...] = jnp.zeros_like(o_ref)

    xb = x_ref[...]
    o_ref[0:1, :] += jnp.sum(xb, axis=0, keepdims=True)
    o_ref[1:2, :] += jnp.sum(xb * xb, axis=0, keepdims=True)


def _col_stats(x, b):
    n, f = x.shape
    return pl.pallas_call(
        _stats_kernel,
        grid=(n // b,),
        in_specs=[_rows(b, f)],
        out_specs=_full((8, f)),
        out_shape=jax.ShapeDtypeStruct((8, f), F32),
        compiler_params=_SEQ,
    )(x)


def _proj_kernel(x_ref, st_ref, g_ref, b_ref, wq_ref, bq_ref, wk_ref, bk_ref,
                 wv_ref, bv_ref, ws_ref, bs_ref, q_ref, k_ref, v_ref, hs_ref,
                 *, n):
    st = st_ref[...]
    mu = st[0:1, :] / n
    var = st[1:2, :] / n - mu * mu
    h = (x_ref[...] - mu) * jax.lax.rsqrt(var + _EPS) * g_ref[...] + b_ref[...]
    q_ref[...] = jnp.dot(h, wq_ref[...], preferred_element_type=F32) + bq_ref[...]
    k_ref[...] = jnp.dot(h, wk_ref[...], preferred_element_type=F32) + bk_ref[...]
    v_ref[...] = jnp.dot(h, wv_ref[...], preferred_element_type=F32) + bv_ref[...]
    hs_ref[...] = jnp.dot(h, ws_ref[...], preferred_element_type=F32) + bs_ref[...]


def _att1_kernel(src_ref, dst_ref, ea_ref, q_ref, k_ref, we_ref, be_ref,
                 lg_ref, mx_ref, ks, qd, *, c, h):
    @pl.when(pl.program_id(0) == 0)
    def _():
        mx_ref[...] = jnp.full_like(mx_ref, -jnp.inf)

    def gather(j, carry):
        s = src_ref[0, j]
        d = dst_ref[0, j]
        ks[pl.ds(j, 1), :] = k_ref[pl.ds(s, 1), :]
        qd[pl.ds(j, 1), :] = q_ref[pl.ds(d, 1), :]
        return carry

    jax.lax.fori_loop(0, c, gather, 0, unroll=8)
    e = jnp.dot(ea_ref[...], we_ref[...], preferred_element_type=F32) + be_ref[...]
    lg = jnp.sum(qd[...] * (ks[...] + e), axis=1, keepdims=True)
    lg_ref[...] = lg / jnp.sqrt(jnp.float32(h))

    def smax(j, carry):
        d = dst_ref[0, j]
        _packed_rmw(mx_ref, d, lg_ref[pl.ds(j, 1), 0:1], jnp.maximum)
        return carry

    jax.lax.fori_loop(0, c, smax, 0, unroll=8)


def _att2_kernel(dst_ref, lg_ref, mx_ref, ex_ref, s_ref, md, *, c):
    @pl.when(pl.program_id(0) == 0)
    def _():
        s_ref[...] = jnp.zeros_like(s_ref)

    def gmax(j, carry):
        d = dst_ref[0, j]
        md[pl.ds(j, 1), 0:1] = _packed_get(mx_ref, d)
        return carry

    jax.lax.fori_loop(0, c, gmax, 0, unroll=8)
    ex_ref[...] = jnp.exp(lg_ref[...] - md[...])

    def sadd(j, carry):
        d = dst_ref[0, j]
        _packed_rmw(s_ref, d, ex_ref[pl.ds(j, 1), 0:1], jnp.add)
        return carry

    jax.lax.fori_loop(0, c, sadd, 0, unroll=8)


def _att3_kernel(src_ref, dst_ref, ea_ref, ex_ref, v_ref, s_ref, we_ref,
                 be_ref, acc_ref, rows, wts, *, c):
    @pl.when(pl.program_id(0) == 0)
    def _():
        acc_ref[...] = jnp.zeros_like(acc_ref)

    def gather(j, carry):
        s = src_ref[0, j]
        d = dst_ref[0, j]
        rows[pl.ds(j, 1), :] = v_ref[pl.ds(s, 1), :]
        wts[pl.ds(j, 1), 0:1] = _packed_get(s_ref, d)
        return carry

    jax.lax.fori_loop(0, c, gather, 0, unroll=8)
    e = jnp.dot(ea_ref[...], we_ref[...], preferred_element_type=F32) + be_ref[...]
    alpha = ex_ref[...] / wts[...]
    rows[...] = (rows[...] + e) * alpha

    def scat(j, carry):
        d = dst_ref[0, j]
        acc_ref[pl.ds(d, 1), :] += rows[pl.ds(j, 1), :]
        return carry

    jax.lax.fori_loop(0, c, scat, 0, unroll=8)


def _addmat_kernel(a_ref, b_ref, w_ref, h_ref, t_ref):
    hv = a_ref[...] + b_ref[...]
    h_ref[...] = hv
    t_ref[...] = jnp.dot(hv, w_ref[...], preferred_element_type=F32)


def _mat_kernel(a_ref, w_ref, t_ref):
    t_ref[...] = jnp.dot(a_ref[...], w_ref[...], preferred_element_type=F32)


def _segsum_kernel(src_ref, dst_ref, t_ref, o_ref, *, c):
    @pl.when(pl.program_id(0) == 0)
    def _():
        o_ref[...] = jnp.zeros_like(o_ref)

    def scat(j, carry):
        s = src_ref[0, j]
        d = dst_ref[0, j]
        o_ref[pl.ds(d, 1), :] += t_ref[pl.ds(s, 1), :]
        return carry

    jax.lax.fori_loop(0, c, scat, 0, unroll=8)


def _segsum(src2d, dst2d, table, n_out, c):
    e = src2d.shape[1]
    nt, hd = table.shape
    return pl.pallas_call(
        functools.partial(_segsum_kernel, c=c),
        grid=(e // c,),
        in_specs=[_lane_smem(c), _lane_smem(c), _full((nt, hd))],
        out_specs=_full((n_out, hd)),
        out_shape=jax.ShapeDtypeStruct((n_out, hd), F32),
        compiler_params=_SEQ,
    )(src2d, dst2d, table)


def _gru_kernel(m_ref, h_ref, wi_ref, wh_ref, bi_ref, bh_ref, o_ref, *, hd):
    gi = jnp.dot(m_ref[...], wi_ref[...], preferred_element_type=F32) + bi_ref[...]
    gh = jnp.dot(h_ref[...], wh_ref[...], preferred_element_type=F32) + bh_ref[...]
    hv = h_ref[...]
    r = jax.nn.sigmoid(gi[:, :hd] + gh[:, :hd])
    z = jax.nn.sigmoid(gi[:, hd:2 * hd] + gh[:, hd:2 * hd])
    cc = jnp.tanh(gi[:, 2 * hd:] + r * gh[:, 2 * hd:])
    o_ref[...] = (1.0 - z) * cc + z * hv


def _gru(m, h, wiT, whT, bi2, bh2, b):
    n, hd = h.shape
    return pl.pallas_call(
        functools.partial(_gru_kernel, hd=hd),
        grid=(n // b,),
        in_specs=[_rows(b, hd), _rows(b, hd), _full(wiT.shape), _full(whT.shape),
                  _full(bi2.shape), _full(bh2.shape)],
        out_specs=_rows(b, hd),
        out_shape=jax.ShapeDtypeStruct((n, hd), F32),
        compiler_params=_SEQ,
    )(m, h, wiT, whT, bi2, bh2)


def _ggc_stack(h, src2d, dst2d, Ws, wiT, whT, bi2, bh2, n_out, c, b,
               skip=None):
    n, hd = h.shape
    for i, W in enumerate(Ws):
        if i == 0 and skip is not None:
            h, t = pl.pallas_call(
                _addmat_kernel,
                grid=(n // b,),
                in_specs=[_rows(b, hd), _rows(b, hd), _full((hd, hd))],
                out_specs=[_rows(b, hd), _rows(b, hd)],
                out_shape=[jax.ShapeDtypeStruct((n, hd), F32),
                           jax.ShapeDtypeStruct((n, hd), F32)],
                compiler_params=_SEQ,
            )(h, skip, W)
        else:
            t = pl.pallas_call(
                _mat_kernel,
                grid=(n // b,),
                in_specs=[_rows(b, hd), _full((hd, hd))],
                out_specs=_rows(b, hd),
                out_shape=jax.ShapeDtypeStruct((n, hd), F32),
                compiler_params=_SEQ,
            )(h, W)
        m = _segsum(src2d, dst2d, t, n_out, c)
        h = _gru(m, h, wiT, whT, bi2, bh2, b)
    return h


def _pool_kernel(idx_ref, h_ref, o_ref, cnt_ref, *, b):
    @pl.when(pl.program_id(0) == 0)
    def _():
        o_ref[...] = jnp.zeros_like(o_ref)
        cnt_ref[...] = jnp.zeros_like(cnt_ref)

    base = pl.program_id(0) * b

    def scat(j, carry):
        d = idx_ref[0, base + j]
        o_ref[pl.ds(d, 1), :] += h_ref[pl.ds(j, 1), :]
        cnt_ref[pl.ds(d, 1), 0:1] += 1.0
        return carry

    jax.lax.fori_loop(0, b, scat, 0, unroll=8)


def _mean_stats_kernel(s_ref, c_ref, o_ref, st_ref):
    @pl.when(pl.program_id(0) == 0)
    def _():
        st_ref[...] = jnp.zeros_like(st_ref)

    hv = s_ref[...] / jnp.maximum(c_ref[...], 1.0)
    o_ref[...] = hv
    st_ref[0:1, :] += jnp.sum(hv, axis=0, keepdims=True)
    st_ref[1:2, :] += jnp.sum(hv * hv, axis=0, keepdims=True)


def _head1_kernel(pid_ref, h_ref, st_ref, g_ref, b_ref, et_ref, wp1_ref,
                  wp2_ref, bp_ref, o_ref, emb, *, b, na):
    st = st_ref[...]
    mu = st[0:1, :] / na
    var = st[1:2, :] / na - mu * mu
    hv = (h_ref[...] - mu) * jax.lax.rsqrt(var + _EPS) * g_ref[...] + b_ref[...]

    base = pl.program_id(0) * b

    def gather(j, carry):
        p = pid_ref[0, base + j]
        emb[pl.ds(j, 1), :] = et_ref[pl.ds(p, 1), :]
        return carry

    jax.lax.fori_loop(0, b, gather, 0, unroll=8)
    o_ref[...] = jax.nn.relu(
        jnp.dot(hv, wp1_ref[...], preferred_element_type=F32)
        + jnp.dot(emb[...], wp2_ref[...], preferred_element_type=F32)
        + bp_ref[...])


def _head2_kernel(i0_ref, i1_ref, h_ref, w1a_ref, w1b_ref, b1_ref, w2_ref,
                  b2_ref, o_ref, rl, rr, *, c):
    def gather(j, carry):
        a = i0_ref[0, j]
        bb = i1_ref[0, j]
        rl[pl.ds(j, 1), :] = h_ref[pl.ds(a, 1), :]
        rr[pl.ds(j, 1), :] = h_ref[pl.ds(bb, 1), :]
        return carry

    jax.lax.fori_loop(0, c, gather, 0, unroll=8)
    hh = jax.nn.relu(
        jnp.dot(rl[...], w1a_ref[...], preferred_element_type=F32)
        + jnp.dot(rr[...], w1b_ref[...], preferred_element_type=F32)
        + b1_ref[...])
    o_ref[...] = (jnp.dot(hh, w2_ref[...], preferred_element_type=F32)
                  + b2_ref[...])


def kernel(x, inner_edge_index, edge_attr, aminoacid_index, protease_id,
           edge_index, gamma1, beta1, Wq, bq, Wk, bk, Wv, bv, We, be, Wskip,
           bskip, Wg1, Wih1, Whh1, bih1, bhh1, gamma2, beta2, emb_table, Wp,
           bp, Wg2, Wih2, Whh2, bih2, bhh2, Wm1, bm1, Wm2, bm2):
    n, f = x.shape
    e_in = inner_edge_index.shape[1]
    na = protease_id.shape[0]
    hd = Wq.shape[1]
    pe = emb_table.shape[1]
    e_out = edge_index.shape[1]
    npairs = e_out // 2

    def _pick(total, prefs):
        for p in prefs:
            if total % p == 0:
                return p
        return total

    nr = (((n + _LANES - 1) // _LANES) + 7) // 8 * 8
    NB = _pick(n, [1000, 500, 250, 200, 100])
    NBa = _pick(na, [1000, 500, 250, 200, 100])
    CE = _pick(e_in, [640, 1280])
    CO = _pick(e_out, [640, 1280])
    CP = _pick(npairs, [3200, 640])

    x = x.astype(F32)
    src = inner_edge_index[0].reshape(1, e_in)
    dst = inner_edge_index[1].reshape(1, e_in)
    osrc = edge_index[0].reshape(1, e_out)
    odst = edge_index[1].reshape(1, e_out)
    i0 = edge_index[0, ::2].reshape(1, npairs)
    i1 = edge_index[1, ::2].reshape(1, npairs)
    aa = aminoacid_index.reshape(1, n)
    pid = protease_id.reshape(1, na)

    row = lambda v: v.reshape(1, -1).astype(F32)

    st1 = _col_stats(x, NB)
    q, k, v, hskip = pl.pallas_call(
        functools.partial(_proj_kernel, n=float(n)),
        grid=(n // NB,),
        in_specs=[_rows(NB, f), _full((8, f)), _full((1, f)), _full((1, f)),
                  _full((f, hd)), _full((1, hd)), _full((f, hd)), _full((1, hd)),
                  _full((f, hd)), _full((1, hd)), _full((f, hd)), _full((1, hd))],
        out_specs=[_rows(NB, hd)] * 4,
        out_shape=[jax.ShapeDtypeStruct((n, hd), F32)] * 4,
        compiler_params=_SEQ,
    )(x, st1, row(gamma1), row(beta1), Wq, row(bq), Wk, row(bk), Wv, row(bv),
      Wskip, row(bskip))

    lg, mx = pl.pallas_call(
        functools.partial(_att1_kernel, c=CE, h=hd),
        grid=(e_in // CE,),
        in_specs=[_lane_smem(CE), _lane_smem(CE),
                  pl.BlockSpec((CE, 5), lambda i: (i, 0)),
                  _full((n, hd)), _full((n, hd)), _full((5, hd)), _full((1, hd))],
        out_specs=[pl.BlockSpec((CE, 1), lambda i: (i, 0)), _full((nr, _LANES))],
        out_shape=[jax.ShapeDtypeStruct((e_in, 1), F32),
                   jax.ShapeDtypeStruct((nr, _LANES), F32)],
        scratch_shapes=[pltpu.VMEM((CE, hd), F32), pltpu.VMEM((CE, hd), F32)],
        compiler_params=_SEQ,
    )(src, dst, edge_attr, q, k, We, row(be))

    ex, ssum = pl.pallas_call(
        functools.partial(_att2_kernel, c=CE),
        grid=(e_in // CE,),
        in_specs=[_lane_smem(CE), pl.BlockSpec((CE, 1), lambda i: (i, 0)),
                  _full((nr, _LANES))],
        out_specs=[pl.BlockSpec((CE, 1), lambda i: (i, 0)),
                   _full((nr, _LANES))],
        out_shape=[jax.ShapeDtypeStruct((e_in, 1), F32),
                   jax.ShapeDtypeStruct((nr, _LANES), F32)],
        scratch_shapes=[pltpu.VMEM((CE, 1), F32)],
        compiler_params=_SEQ,
    )(dst, lg, mx)

    agg = pl.pallas_call(
        functools.partial(_att3_kernel, c=CE),
        grid=(e_in // CE,),
        in_specs=[_lane_smem(CE), _lane_smem(CE),
                  pl.BlockSpec((CE, 5), lambda i: (i, 0)),
                  pl.BlockSpec((CE, 1), lambda i: (i, 0)), _full((n, hd)),
                  _full((nr, _LANES)), _full((5, hd)), _full((1, hd))],
        out_specs=_full((n, hd)),
        out_shape=jax.ShapeDtypeStruct((n, hd), F32),
        scratch_shapes=[pltpu.VMEM((CE, hd), F32), pltpu.VMEM((CE, 1), F32)],
        compiler_params=_SEQ,
    )(src, dst, edge_attr, ex, v, ssum, We, row(be))

    h = _ggc_stack(agg, src, dst, [Wg1[i] for i in range(Wg1.shape[0])],
                   Wih1.T, Whh1.T, row(bih1), row(bhh1), n, CE, NB,
                   skip=hskip)

    sums, cnt = pl.pallas_call(
        functools.partial(_pool_kernel, b=NB),
        grid=(n // NB,),
        in_specs=[pl.BlockSpec((1, n), lambda i: (0, 0), memory_space=pltpu.SMEM),
                  _rows(NB, hd)],
        out_specs=[_full((na, hd)), _full((na, 1))],
        out_shape=[jax.ShapeDtypeStruct((na, hd), F32),
                   jax.ShapeDtypeStruct((na, 1), F32)],
        compiler_params=_SEQ,
    )(aa, h)

    hp, st2 = pl.pallas_call(
        _mean_stats_kernel,
        grid=(na // NBa,),
        in_specs=[_rows(NBa, hd), pl.BlockSpec((NBa, 1), lambda i: (i, 0))],
        out_specs=[_rows(NB, hd), _full((8, hd))],
        out_shape=[jax.ShapeDtypeStruct((na, hd), F32),
                   jax.ShapeDtypeStruct((8, hd), F32)],
        compiler_params=_SEQ,
    )(sums, cnt)

    h2 = pl.pallas_call(
        functools.partial(_head1_kernel, b=NBa, na=float(na)),
        grid=(na // NBa,),
        in_specs=[pl.BlockSpec((1, na), lambda i: (0, 0), memory_space=pltpu.SMEM),
                  _rows(NBa, hd), _full((8, hd)),
                  _full((1, hd)), _full((1, hd)), _full(emb_table.shape),
                  _full((hd, hd)), _full((pe, hd)), _full((1, hd))],
        out_specs=_rows(NBa, hd),
        out_shape=jax.ShapeDtypeStruct((na, hd), F32),
        scratch_shapes=[pltpu.VMEM((NBa, pe), F32)],
        compiler_params=_SEQ,
    )(pid, hp, st2, row(gamma2), row(beta2), emb_table, Wp[:hd], Wp[hd:],
      row(bp))

    h2 = _ggc_stack(h2, osrc, odst, [Wg2[i] for i in range(Wg2.shape[0])],
                    Wih2.T, Whh2.T, row(bih2), row(bhh2), na, CO, NBa)

    out = pl.pallas_call(
        functools.partial(_head2_kernel, c=CP),
        grid=(npairs // CP,),
        in_specs=[_lane_smem(CP), _lane_smem(CP), _full((na, hd)),
                  _full((hd, hd)), _full((hd, hd)), _full((1, hd)),
                  _full((hd, 1)), _full((1, 1))],
        out_specs=pl.BlockSpec((CP, 1), lambda i: (i, 0)),
        out_shape=jax.ShapeDtypeStruct((npairs, 1), F32),
        scratch_shapes=[pltpu.VMEM((CP, hd), F32), pltpu.VMEM((CP, hd), F32)],
        compiler_params=_SEQ,
    )(i0, i1, h2, Wm1[:hd], Wm1[hd:], row(bm1), Wm2, row(bm2))

    return out.reshape(-1)
